# baseline (speedup 1.0000x reference)
import jax
import jax.numpy as jnp
from jax.experimental import pallas as pl
from jax.experimental.pallas import tpu as pltpu

_B, _H, _W = 8, 512, 512
_N = _H * _W
_BB = 2
_STEPS = _B // _BB


def _bce_kernel(p_ref, t_ref, out_ref):
    p = p_ref[...]
    t = t_ref[...]
    sel = jnp.where(t == 1.0, p, 1.0 - p)
    v = jnp.maximum(jnp.log(sel), -100.0)
    totalv = jnp.sum(v, axis=(1, 2))
    s1v = jnp.sum(t * v, axis=(1, 2))
    c1v = jnp.sum(t, axis=(1, 2))
    s0v = totalv - s1v
    contrib = jnp.sum(c1v * s1v + (_N - c1v) * s0v) * (
        -1.0 / (_N * float(_N) * _B)
    )
    out_ref[:, :, :] = jnp.full((1, 1, 1), contrib)


def kernel(predict, target):
    out = pl.pallas_call(
        _bce_kernel,
        grid=(_STEPS,),
        in_specs=[
            pl.BlockSpec((_BB, _H, _W), lambda b: (b, 0, 0)),
            pl.BlockSpec((_BB, _H, _W), lambda b: (b, 0, 0)),
        ],
        out_specs=pl.BlockSpec((1, 1, 1), lambda b: (b, 0, 0)),
        out_shape=jax.ShapeDtypeStruct((_STEPS, 1, 1), jnp.float32),
        compiler_params=pltpu.CompilerParams(
            dimension_semantics=("parallel",)
        ),
    )(predict, target)
    return jnp.sum(out)


# speedup vs baseline: 1.1748x; 1.1748x over previous
import jax
import jax.numpy as jnp
from jax.experimental import pallas as pl

_B, _H, _W = 8, 512, 512
_N = _H * _W
_HB = 2
_STEPS = _B // (2 * _HB)


def _partial(p, t):
    sel = jnp.where(t == 1.0, p, 1.0 - p)
    v = jnp.maximum(jnp.log(sel), -100.0)
    totalv = jnp.sum(v, axis=(1, 2))
    s1v = jnp.sum(t * v, axis=(1, 2))
    c1v = jnp.sum(t, axis=(1, 2))
    s0v = totalv - s1v
    return jnp.sum(c1v * s1v + (_N - c1v) * s0v)


def _bce_kernel(pa_ref, pb_ref, ta_ref, tb_ref, out_ref):
    b = pl.program_id(0)
    contrib = (_partial(pa_ref[...], ta_ref[...]) +
               _partial(pb_ref[...], tb_ref[...])) * (
        -1.0 / (_N * float(_N) * _B)
    )

    @pl.when(b == 0)
    def _init():
        out_ref[:, :] = jnp.zeros((1, 1), jnp.float32)

    out_ref[:, :] += jnp.full((1, 1), contrib)


def kernel(predict, target):
    blk_a = pl.BlockSpec((_HB, _H, _W), lambda b: (2 * b, 0, 0))
    blk_b = pl.BlockSpec((_HB, _H, _W), lambda b: (2 * b + 1, 0, 0))
    out = pl.pallas_call(
        _bce_kernel,
        grid=(_STEPS,),
        in_specs=[blk_a, blk_b, blk_a, blk_b],
        out_specs=pl.BlockSpec((1, 1), lambda b: (0, 0)),
        out_shape=jax.ShapeDtypeStruct((1, 1), jnp.float32),
    )(predict, predict, target, target)
    return out[0, 0]
